# baseline (speedup 1.0000x reference)
import functools

import jax
import jax.numpy as jnp
import numpy as np
from jax.experimental import pallas as pl
from jax.experimental.pallas import tpu as pltpu

_BLANK = 0
_LN2 = 0.6931471805599453


def _shift_right(x, k, fill=0.0):
    pad = jnp.full(x.shape[:-1] + (k,), fill, x.dtype)
    return jnp.concatenate([pad, x[..., :-k]], axis=-1)


def _cumsum_lanes(x):
    n = x.shape[-1]
    k = 1
    while k < n:
        x = x + _shift_right(x, k)
        k *= 2
    return x


def _scan_pair(m, s):
    n = m.shape[-1]
    k = 1
    while k < n:
        m_sh = _shift_right(m, k, -1e30)
        s_sh = _shift_right(s, k, 0.0)
        big = m >= m_sh
        hi = jnp.maximum(m, m_sh)
        lo = jnp.minimum(m, m_sh)
        e = jnp.exp(lo - hi)
        s_hi = jnp.where(big, s, s_sh)
        s_lo = jnp.where(big, s_sh, s)
        s = s_hi + s_lo * e
        m = hi
        k *= 2
    return m, s


def _renorm(m, s):
    bits = jax.lax.bitcast_convert_type(s, jnp.int32)
    eb = jax.lax.shift_right_logical(bits, 23) - 127
    mant = jax.lax.bitcast_convert_type(
        (bits & jnp.int32(0x007FFFFF)) | jnp.int32(0x3F800000), jnp.float32)
    return m + eb.astype(jnp.float32) * _LN2, mant


def _dense_kernel(mask_ref, wsum_ref, wsel_ref, x_ref, c_ref, d_ref, dcarry_ref,
                  *, bsz, usz):
    x = x_ref[...]
    wsum = wsum_ref[...]
    wsel = wsel_ref[...]
    c_parts = []
    d_parts = []
    for b in range(bsz):
        xb = x[b]
        e = jnp.exp(xb).astype(jnp.bfloat16)
        mb = (xb * mask_ref[b:b + 1, :]).astype(jnp.bfloat16)
        s = jax.lax.dot(e, wsum, preferred_element_type=jnp.float32)
        gl = jax.lax.dot(mb, wsel, preferred_element_type=jnp.float32)
        lse = jnp.log(s[:, :usz])
        lab = gl[:, :usz] - lse
        blank = gl[:, usz:] - lse
        ll = _shift_right(_cumsum_lanes(lab), 1)
        d = ll + blank
        d_prev = jnp.concatenate([dcarry_ref[b:b + 1, :], d[:-1, :]], axis=0)
        c_parts.append(d_prev - ll)
        d_parts.append(d)
        dcarry_ref[b:b + 1, :] = d[-1:, :]
    c_ref[...] = jnp.stack(c_parts, axis=1)
    d_ref[...] = jnp.stack(d_parts, axis=1)


def _dp_kernel(tl_ref, ul_ref, c_ref, d_ref, out_ref, *, unroll):
    T = c_ref.shape[0]
    bsz = c_ref.shape[1]
    lane = jax.lax.broadcasted_iota(jnp.int32, (bsz, c_ref.shape[2]), 1)
    tl = tl_ref[...]
    umask = lane == ul_ref[...]
    zero = jnp.zeros((bsz, c_ref.shape[2]), jnp.float32)
    one = jnp.ones_like(zero)

    cap0 = (tl == 0) & umask
    acc_m = jnp.where(cap0, d_ref[0], zero)
    acc_s = one

    def body(j, carry):
        m, s, acc_m, acc_s = carry
        t0 = 1 + j * unroll
        for i in range(unroll):
            t = t0 + i
            m, s = _scan_pair(m + c_ref[t], s)
            cap = (tl == t) & umask
            acc_m = jnp.where(cap, m + d_ref[t], acc_m)
            acc_s = jnp.where(cap, s, acc_s)
        m, s = _renorm(m, s)
        return m, s, acc_m, acc_s

    nblocks = (T - 1) // unroll
    _, _, acc_m, acc_s = jax.lax.fori_loop(
        0, nblocks, body, (zero, one, acc_m, acc_s))
    loss = -jnp.sum(acc_m + jnp.log(acc_s)) / bsz
    out_ref[...] = jnp.broadcast_to(loss, (1, 1))


def kernel(logits, logit_lens, y, y_lens, batch_offset, max_f_len):
    B, T, U, V = logits.shape
    TT = min(64, T)
    unroll = 7 if (T - 1) % 7 == 0 else 1
    y = y.astype(jnp.int32)
    y_pad = jnp.concatenate([y, jnp.zeros((B, 1), jnp.int32)], axis=1)
    tl = (logit_lens.astype(jnp.int32) - 1).reshape(B, 1)
    ul = y_lens.astype(jnp.int32).reshape(B, 1)

    v_idx = jnp.arange(V, dtype=jnp.int32)
    onehot = (y_pad[:, :, None] == v_idx[None, None, :]) | (v_idx[None, None, :] == _BLANK)
    mask = onehot.reshape(B, U * V).astype(jnp.float32)

    r_u = np.arange(U * V) // V
    r_v = np.arange(U * V) % V
    c = np.arange(2 * U)
    wsum_np = (r_u[:, None] == (c[None, :] % U)).astype(np.float32)
    wsel_np = (((r_u[:, None] == c[None, :]) & (r_v[:, None] >= 1))
               | ((r_u[:, None] == c[None, :] - U) & (r_v[:, None] == _BLANK)))
    wsum = jnp.asarray(wsum_np, dtype=jnp.bfloat16)
    wsel = jnp.asarray(wsel_np.astype(np.float32), dtype=jnp.bfloat16)

    x3 = logits.reshape(B, T, U * V)

    c_arr, d_arr = pl.pallas_call(
        functools.partial(_dense_kernel, bsz=B, usz=U),
        grid=(T // TT,),
        in_specs=[
            pl.BlockSpec((B, U * V), lambda i: (0, 0)),
            pl.BlockSpec((U * V, 2 * U), lambda i: (0, 0)),
            pl.BlockSpec((U * V, 2 * U), lambda i: (0, 0)),
            pl.BlockSpec((B, TT, U * V), lambda i: (0, i, 0)),
        ],
        out_specs=[
            pl.BlockSpec((TT, B, U), lambda i: (i, 0, 0)),
            pl.BlockSpec((TT, B, U), lambda i: (i, 0, 0)),
        ],
        out_shape=[
            jax.ShapeDtypeStruct((T, B, U), jnp.float32),
            jax.ShapeDtypeStruct((T, B, U), jnp.float32),
        ],
        scratch_shapes=[pltpu.VMEM((B, U), jnp.float32)],
    )(mask, wsum, wsel, x3)

    out = pl.pallas_call(
        functools.partial(_dp_kernel, unroll=unroll),
        out_shape=jax.ShapeDtypeStruct((1, 1), jnp.float32),
    )(tl, ul, c_arr, d_arr)
    return out[0, 0]
